# baseline (speedup 1.0000x reference)
import jax
import jax.numpy as jnp
from jax import lax
from jax.experimental import pallas as pl
from jax.experimental.pallas import tpu as pltpu

_D_MODEL = 1024
_D_SPARSE = 8192
_K = 64
_N_TOK = 4096

_NB_ENC = 512
_TB_SEL = 128


def _encode_body(x_ref, wet_ref, eb_ref, db_ref, pre_ref):
    xp = x_ref[...] - db_ref[...]
    acts = jnp.dot(xp, wet_ref[...], preferred_element_type=jnp.float32)
    pre_ref[...] = jnp.maximum(acts + eb_ref[...], 0.0)


def _select_body(pre_ref, wdt_ref, x_ref, db_ref, hid_ref, out_ref,
                 l2_ref, rec_ref):
    pre = pre_ref[...]
    tb = pre.shape[0]

    def _count_ge(t_bits):
        t_f = lax.bitcast_convert_type(t_bits, jnp.float32)
        return jnp.sum((pre >= t_f).astype(jnp.float32), axis=1, keepdims=True)

    def _bisect_bits(it, carry):
        lo, hi, cnt_lo, cnt_hi = carry
        mid = lo + lax.shift_right_logical(hi - lo, 1)
        cnt = _count_ge(mid)
        take = cnt >= float(_K)
        lo = jnp.where(take, mid, lo)
        hi = jnp.where(take, hi, mid)
        cnt_lo = jnp.where(take, cnt, cnt_lo)
        cnt_hi = jnp.where(take, cnt_hi, cnt)
        return lo, hi, cnt_lo, cnt_hi

    lo0 = jnp.zeros((tb, 1), jnp.int32)
    hi0 = jnp.full((tb, 1), 0x7F800000, jnp.int32)
    c_lo0 = jnp.full((tb, 1), float(_D_SPARSE), jnp.float32)
    c_hi0 = jnp.zeros((tb, 1), jnp.float32)
    lo, hi, n_ge, n_gt = lax.fori_loop(
        0, 31, _bisect_bits, (lo0, hi0, c_lo0, c_hi0))
    t_f = lax.bitcast_convert_type(lo, jnp.float32)
    budget = float(_K) - n_gt
    n_ties = n_ge - n_gt

    row_ok = jnp.logical_or(n_ties == budget, lo == 0)
    hid = jnp.where(pre >= t_f, pre, 0.0)
    hid_ref[...] = hid

    @pl.when(jnp.logical_not(jnp.all(row_ok)))
    def _slow_tie_path():
        idx = lax.broadcasted_iota(jnp.int32, pre.shape, 1)
        tie = pre == t_f

        def _g(cut):
            m = jnp.logical_and(tie, idx <= cut)
            return jnp.sum(m.astype(jnp.float32), axis=1, keepdims=True)

        def _bisect_idx(it, carry):
            lo2, hi2 = carry
            mid = lo2 + lax.shift_right_logical(hi2 - lo2, 1)
            ok = _g(mid) >= budget
            hi2 = jnp.where(ok, mid, hi2)
            lo2 = jnp.where(ok, lo2, mid)
            return lo2, hi2

        lo2 = jnp.full((tb, 1), -1, jnp.int32)
        hi2 = jnp.full((tb, 1), _D_SPARSE - 1, jnp.int32)
        lo2, hi2 = lax.fori_loop(0, 13, _bisect_idx, (lo2, hi2))
        keep = jnp.logical_or(pre > t_f,
                              jnp.logical_and(tie, idx <= hi2))
        hid_ref[...] = jnp.where(keep, pre, 0.0)

    i = pl.program_id(0)
    ni = pl.num_programs(0)
    acc = jnp.dot(hid_ref[...].astype(jnp.bfloat16), wdt_ref[...],
                  preferred_element_type=jnp.float32)
    sae = acc + db_ref[...]
    out_ref[...] = sae
    e = sae - x_ref[...]
    partial = jnp.sum(e * e, axis=(0, 1), keepdims=True)

    @pl.when(i == 0)
    def _set():
        l2_ref[...] = partial

    @pl.when(i != 0)
    def _add():
        l2_ref[...] = l2_ref[...] + partial

    @pl.when(i == ni - 1)
    def _rec():
        rec_ref[...] = l2_ref[...] * (1.0 / float(_N_TOK * _D_MODEL))


def kernel(x, W_enc, enc_bias, W_dec, dec_bias):
    wet = W_enc.T
    wdt = W_dec.astype(jnp.bfloat16).T
    eb = enc_bias.reshape(1, _D_SPARSE)
    db = dec_bias.reshape(1, _D_MODEL)

    pre = pl.pallas_call(
        _encode_body,
        grid=(_D_SPARSE // _NB_ENC,),
        in_specs=[
            pl.BlockSpec((_N_TOK, _D_MODEL), lambda j: (0, 0)),
            pl.BlockSpec((_D_MODEL, _NB_ENC), lambda j: (0, j)),
            pl.BlockSpec((1, _NB_ENC), lambda j: (0, j)),
            pl.BlockSpec((1, _D_MODEL), lambda j: (0, 0)),
        ],
        out_specs=pl.BlockSpec((_N_TOK, _NB_ENC), lambda j: (0, j)),
        out_shape=jax.ShapeDtypeStruct((_N_TOK, _D_SPARSE), jnp.float32),
    )(x, wet, eb, db)

    hidden, sae, l2, rec = pl.pallas_call(
        _select_body,
        grid=(_N_TOK // _TB_SEL,),
        in_specs=[
            pl.BlockSpec((_TB_SEL, _D_SPARSE), lambda i: (i, 0)),
            pl.BlockSpec((_D_SPARSE, _D_MODEL), lambda i: (0, 0)),
            pl.BlockSpec((_TB_SEL, _D_MODEL), lambda i: (i, 0)),
            pl.BlockSpec((1, _D_MODEL), lambda i: (0, 0)),
        ],
        out_specs=[
            pl.BlockSpec((_TB_SEL, _D_SPARSE), lambda i: (i, 0)),
            pl.BlockSpec((_TB_SEL, _D_MODEL), lambda i: (i, 0)),
            pl.BlockSpec((1, 1), lambda i: (0, 0)),
            pl.BlockSpec((1, 1), lambda i: (0, 0)),
        ],
        out_shape=[
            jax.ShapeDtypeStruct((_N_TOK, _D_SPARSE), jnp.float32),
            jax.ShapeDtypeStruct((_N_TOK, _D_MODEL), jnp.float32),
            jax.ShapeDtypeStruct((1, 1), jnp.float32),
            jax.ShapeDtypeStruct((1, 1), jnp.float32),
        ],
    )(pre, wdt, x, db)

    return sae, hidden, l2[0, 0], rec[0, 0]


# speedup vs baseline: 1.0529x; 1.0529x over previous
import jax
import jax.numpy as jnp
from jax import lax
from jax.experimental import pallas as pl
from jax.experimental.pallas import tpu as pltpu

_D_MODEL = 1024
_D_SPARSE = 8192
_K = 64
_N_TOK = 4096

_NB_ENC = 512
_TB_SEL = 256
_TB_B = 512


def _encode_body(x_ref, wet_ref, eb_ref, db_ref, pre_ref):
    xp = x_ref[...] - db_ref[...]
    acts = jnp.dot(xp, wet_ref[...], preferred_element_type=jnp.float32)
    pre_ref[...] = jnp.maximum(acts + eb_ref[...], 0.0)


def _select_body(pre_ref, hid_ref, hid16_ref):
    pre = pre_ref[...]
    tb = pre.shape[0]

    def _count_ge(t_bits):
        t_f = lax.bitcast_convert_type(t_bits, jnp.float32)
        return jnp.sum((pre >= t_f).astype(jnp.float32), axis=1, keepdims=True)

    def _bisect_bits(it, carry):
        lo, hi, cnt_lo, cnt_hi = carry
        mid = lo + lax.shift_right_logical(hi - lo, 1)
        cnt = _count_ge(mid)
        take = cnt >= float(_K)
        lo = jnp.where(take, mid, lo)
        hi = jnp.where(take, hi, mid)
        cnt_lo = jnp.where(take, cnt, cnt_lo)
        cnt_hi = jnp.where(take, cnt_hi, cnt)
        return lo, hi, cnt_lo, cnt_hi

    lo0 = jnp.zeros((tb, 1), jnp.int32)
    hi0 = jnp.full((tb, 1), 0x7F800000, jnp.int32)
    c_lo0 = jnp.full((tb, 1), float(_D_SPARSE), jnp.float32)
    c_hi0 = jnp.zeros((tb, 1), jnp.float32)
    lo, hi, n_ge, n_gt = lax.fori_loop(
        0, 31, _bisect_bits, (lo0, hi0, c_lo0, c_hi0))
    t_f = lax.bitcast_convert_type(lo, jnp.float32)
    budget = float(_K) - n_gt
    n_ties = n_ge - n_gt

    row_ok = jnp.logical_or(n_ties == budget, lo == 0)
    hid = jnp.where(pre >= t_f, pre, 0.0)
    hid_ref[...] = hid
    hid16_ref[...] = hid.astype(jnp.bfloat16)

    @pl.when(jnp.logical_not(jnp.all(row_ok)))
    def _slow_tie_path():
        idx = lax.broadcasted_iota(jnp.int32, pre.shape, 1)
        tie = pre == t_f

        def _g(cut):
            m = jnp.logical_and(tie, idx <= cut)
            return jnp.sum(m.astype(jnp.float32), axis=1, keepdims=True)

        def _bisect_idx(it, carry):
            lo2, hi2 = carry
            mid = lo2 + lax.shift_right_logical(hi2 - lo2, 1)
            ok = _g(mid) >= budget
            hi2 = jnp.where(ok, mid, hi2)
            lo2 = jnp.where(ok, lo2, mid)
            return lo2, hi2

        lo2 = jnp.full((tb, 1), -1, jnp.int32)
        hi2 = jnp.full((tb, 1), _D_SPARSE - 1, jnp.int32)
        lo2, hi2 = lax.fori_loop(0, 13, _bisect_idx, (lo2, hi2))
        keep = jnp.logical_or(pre > t_f,
                              jnp.logical_and(tie, idx <= hi2))
        hid2 = jnp.where(keep, pre, 0.0)
        hid_ref[...] = hid2
        hid16_ref[...] = hid2.astype(jnp.bfloat16)


def _decode_body(hid16_ref, wdt_ref, x_ref, db_ref, out_ref, l2_ref, rec_ref):
    i = pl.program_id(0)
    ni = pl.num_programs(0)

    acc = jnp.dot(hid16_ref[...], wdt_ref[...],
                  preferred_element_type=jnp.float32)
    sae = acc + db_ref[...]
    out_ref[...] = sae
    e = sae - x_ref[...]
    partial = jnp.sum(e * e, axis=(0, 1), keepdims=True)

    @pl.when(i == 0)
    def _set():
        l2_ref[...] = partial

    @pl.when(i != 0)
    def _add():
        l2_ref[...] = l2_ref[...] + partial

    @pl.when(i == ni - 1)
    def _rec():
        rec_ref[...] = l2_ref[...] * (1.0 / float(_N_TOK * _D_MODEL))


def kernel(x, W_enc, enc_bias, W_dec, dec_bias):
    wet = W_enc.T
    wdt = W_dec.astype(jnp.bfloat16).T
    eb = enc_bias.reshape(1, _D_SPARSE)
    db = dec_bias.reshape(1, _D_MODEL)

    pre = pl.pallas_call(
        _encode_body,
        grid=(_D_SPARSE // _NB_ENC,),
        in_specs=[
            pl.BlockSpec((_N_TOK, _D_MODEL), lambda j: (0, 0)),
            pl.BlockSpec((_D_MODEL, _NB_ENC), lambda j: (0, j)),
            pl.BlockSpec((1, _NB_ENC), lambda j: (0, j)),
            pl.BlockSpec((1, _D_MODEL), lambda j: (0, 0)),
        ],
        out_specs=pl.BlockSpec((_N_TOK, _NB_ENC), lambda j: (0, j)),
        out_shape=jax.ShapeDtypeStruct((_N_TOK, _D_SPARSE), jnp.float32),
    )(x, wet, eb, db)

    hidden, hidden16 = pl.pallas_call(
        _select_body,
        grid=(_N_TOK // _TB_SEL,),
        in_specs=[
            pl.BlockSpec((_TB_SEL, _D_SPARSE), lambda i: (i, 0)),
        ],
        out_specs=[
            pl.BlockSpec((_TB_SEL, _D_SPARSE), lambda i: (i, 0)),
            pl.BlockSpec((_TB_SEL, _D_SPARSE), lambda i: (i, 0)),
        ],
        out_shape=[
            jax.ShapeDtypeStruct((_N_TOK, _D_SPARSE), jnp.float32),
            jax.ShapeDtypeStruct((_N_TOK, _D_SPARSE), jnp.bfloat16),
        ],
    )(pre)

    sae, l2, rec = pl.pallas_call(
        _decode_body,
        grid=(_N_TOK // _TB_B,),
        in_specs=[
            pl.BlockSpec((_TB_B, _D_SPARSE), lambda i: (i, 0)),
            pl.BlockSpec((_D_SPARSE, _D_MODEL), lambda i: (0, 0)),
            pl.BlockSpec((_TB_B, _D_MODEL), lambda i: (i, 0)),
            pl.BlockSpec((1, _D_MODEL), lambda i: (0, 0)),
        ],
        out_specs=[
            pl.BlockSpec((_TB_B, _D_MODEL), lambda i: (i, 0)),
            pl.BlockSpec((1, 1), lambda i: (0, 0)),
            pl.BlockSpec((1, 1), lambda i: (0, 0)),
        ],
        out_shape=[
            jax.ShapeDtypeStruct((_N_TOK, _D_MODEL), jnp.float32),
            jax.ShapeDtypeStruct((1, 1), jnp.float32),
            jax.ShapeDtypeStruct((1, 1), jnp.float32),
        ],
    )(hidden16, wdt, x, db)

    return sae, hidden, l2[0, 0], rec[0, 0]
